# baseline (speedup 1.0000x reference)
import functools
import math

import jax
import jax.numpy as jnp
from jax.experimental import pallas as pl
from jax.experimental.pallas import tpu as pltpu

_EPS = 1e-5
_INV_SQRT2 = 1.0 / math.sqrt(2.0)
_VMEM_LIMIT = 48 * 1024 * 1024
_TM1 = 16384
_TM = 4096


def _pack_w(wr_ref, wi_ref):
    return jnp.concatenate(
        [jnp.concatenate([wr_ref[...], wi_ref[...]], axis=1),
         jnp.concatenate([-wi_ref[...], wr_ref[...]], axis=1)], axis=0)


def _stats_kernel(xr_ref, xi_ref, wr_ref, wi_ref, sum_ref, sq_ref):
    xp = jnp.concatenate([xr_ref[...], xi_ref[...]], axis=1)
    y = jnp.dot(xp, _pack_w(wr_ref, wi_ref),
                preferred_element_type=jnp.float32)
    sum_ref[...] = jnp.sum(y, axis=0, keepdims=True)[None]
    sq_ref[...] = jnp.sum(y * y, axis=0, keepdims=True)[None]


def _bn_gelu_kernel(xr_ref, xi_ref, wr_ref, wi_ref, sum_ref, sq_ref,
                    gr_ref, gi_ref, br_ref, bi_ref, or_ref, oi_ref,
                    *, inv_m, cout):
    xp = jnp.concatenate([xr_ref[...], xi_ref[...]], axis=1)
    y = jnp.dot(xp, _pack_w(wr_ref, wi_ref),
                preferred_element_type=jnp.float32)
    mean = jnp.sum(sum_ref[...], axis=0) * inv_m
    var = jnp.maximum(jnp.sum(sq_ref[...], axis=0) * inv_m - mean * mean, 0.0)
    g = jnp.concatenate([gr_ref[...], gi_ref[...]], axis=1)
    b = jnp.concatenate([br_ref[...], bi_ref[...]], axis=1)
    scale = jax.lax.rsqrt(var + _EPS) * g
    shift = b - mean * scale
    z = y * scale + shift
    o = 0.5 * z * (1.0 + jax.lax.erf(z * _INV_SQRT2))
    or_ref[...] = o[:, :cout]
    oi_ref[...] = o[:, cout:]


def kernel(x_real, x_imag, wr, wi, br, bi, gr, betar, gi, betai):
    N, Cin, H, W = x_real.shape
    Cout = wr.shape[1]
    M = N * H * W
    C2 = 2 * Cout
    nt1 = M // _TM1
    nt = M // _TM

    xr = x_real.transpose(0, 2, 3, 1).reshape(M, Cin)
    xi = x_imag.transpose(0, 2, 3, 1).reshape(M, Cin)

    x1_spec = pl.BlockSpec((_TM1, Cin), lambda i: (i, 0))
    x_spec = pl.BlockSpec((_TM, Cin), lambda i: (i, 0))
    w_spec = pl.BlockSpec((Cin, Cout), lambda i: (0, 0))
    row_spec = pl.BlockSpec((1, Cout), lambda i: (0, 0))
    stat_spec = pl.BlockSpec((1, 1, C2), lambda i: (i, 0, 0))

    ysum, ysq = pl.pallas_call(
        _stats_kernel,
        grid=(nt1,),
        in_specs=[x1_spec, x1_spec, w_spec, w_spec],
        out_specs=[stat_spec, stat_spec],
        out_shape=[
            jax.ShapeDtypeStruct((nt1, 1, C2), jnp.float32),
            jax.ShapeDtypeStruct((nt1, 1, C2), jnp.float32),
        ],
        compiler_params=pltpu.CompilerParams(
            dimension_semantics=("parallel",),
            vmem_limit_bytes=_VMEM_LIMIT),
    )(xr, xi, wr, wi)

    allstat_spec = pl.BlockSpec((nt1, 1, C2), lambda i: (0, 0, 0))
    out_spec = pl.BlockSpec((_TM, Cout), lambda i: (i, 0))

    o_real, o_imag = pl.pallas_call(
        functools.partial(_bn_gelu_kernel, inv_m=1.0 / M, cout=Cout),
        grid=(nt,),
        in_specs=[x_spec, x_spec, w_spec, w_spec, allstat_spec, allstat_spec,
                  row_spec, row_spec, row_spec, row_spec],
        out_specs=[out_spec, out_spec],
        out_shape=[
            jax.ShapeDtypeStruct((M, Cout), jnp.float32),
            jax.ShapeDtypeStruct((M, Cout), jnp.float32),
        ],
        compiler_params=pltpu.CompilerParams(
            dimension_semantics=("parallel",),
            vmem_limit_bytes=_VMEM_LIMIT),
    )(xr, xi, wr, wi, ysum, ysq, gr, gi, betar, betai)

    def to_nchw(v):
        return v.reshape(N, H, W, Cout).transpose(0, 3, 1, 2)

    return {"real": to_nchw(o_real), "imag": to_nchw(o_imag)}


# speedup vs baseline: 1.0289x; 1.0289x over previous
import functools
import math

import jax
import jax.numpy as jnp
from jax.experimental import pallas as pl
from jax.experimental.pallas import tpu as pltpu

_EPS = 1e-5
_INV_SQRT2 = 1.0 / math.sqrt(2.0)
_VMEM_LIMIT = 48 * 1024 * 1024
_TM1 = 16384
_TM = 8192


def _pack_w(wr_ref, wi_ref):
    return jnp.concatenate(
        [jnp.concatenate([wr_ref[...], wi_ref[...]], axis=1),
         jnp.concatenate([-wi_ref[...], wr_ref[...]], axis=1)], axis=0)


def _stats_kernel(xr_ref, xi_ref, wr_ref, wi_ref, sum_ref, sq_ref):
    xp = jnp.concatenate([xr_ref[...], xi_ref[...]], axis=1)
    y = jnp.dot(xp, _pack_w(wr_ref, wi_ref),
                preferred_element_type=jnp.float32)
    sum_ref[...] = jnp.sum(y, axis=0, keepdims=True)[None]
    sq_ref[...] = jnp.sum(y * y, axis=0, keepdims=True)[None]


def _bn_gelu_kernel(xr_ref, xi_ref, wr_ref, wi_ref, sum_ref, sq_ref,
                    gr_ref, gi_ref, br_ref, bi_ref, or_ref, oi_ref,
                    *, inv_m, cout):
    xp = jnp.concatenate([xr_ref[...], xi_ref[...]], axis=1)
    y = jnp.dot(xp, _pack_w(wr_ref, wi_ref),
                preferred_element_type=jnp.float32)
    mean = jnp.sum(sum_ref[...], axis=0) * inv_m
    var = jnp.maximum(jnp.sum(sq_ref[...], axis=0) * inv_m - mean * mean, 0.0)
    g = jnp.concatenate([gr_ref[...], gi_ref[...]], axis=1)
    b = jnp.concatenate([br_ref[...], bi_ref[...]], axis=1)
    scale = jax.lax.rsqrt(var + _EPS) * g
    shift = b - mean * scale
    z = y * scale + shift
    o = 0.5 * z * (1.0 + jax.lax.erf(z * _INV_SQRT2))
    or_ref[...] = o[:, :cout]
    oi_ref[...] = o[:, cout:]


def kernel(x_real, x_imag, wr, wi, br, bi, gr, betar, gi, betai):
    N, Cin, H, W = x_real.shape
    Cout = wr.shape[1]
    M = N * H * W
    C2 = 2 * Cout
    nt1 = M // _TM1
    nt = M // _TM

    xr = x_real.transpose(0, 2, 3, 1).reshape(M, Cin)
    xi = x_imag.transpose(0, 2, 3, 1).reshape(M, Cin)

    x1_spec = pl.BlockSpec((_TM1, Cin), lambda i: (i, 0))
    x_spec = pl.BlockSpec((_TM, Cin), lambda i: (i, 0))
    w_spec = pl.BlockSpec((Cin, Cout), lambda i: (0, 0))
    row_spec = pl.BlockSpec((1, Cout), lambda i: (0, 0))
    stat_spec = pl.BlockSpec((1, 1, C2), lambda i: (i, 0, 0))

    ysum, ysq = pl.pallas_call(
        _stats_kernel,
        grid=(nt1,),
        in_specs=[x1_spec, x1_spec, w_spec, w_spec],
        out_specs=[stat_spec, stat_spec],
        out_shape=[
            jax.ShapeDtypeStruct((nt1, 1, C2), jnp.float32),
            jax.ShapeDtypeStruct((nt1, 1, C2), jnp.float32),
        ],
        compiler_params=pltpu.CompilerParams(
            dimension_semantics=("parallel",),
            vmem_limit_bytes=_VMEM_LIMIT),
    )(xr, xi, wr, wi)

    allstat_spec = pl.BlockSpec((nt1, 1, C2), lambda i: (0, 0, 0))
    out_spec = pl.BlockSpec((_TM, Cout), lambda i: (i, 0))

    o_real, o_imag = pl.pallas_call(
        functools.partial(_bn_gelu_kernel, inv_m=1.0 / M, cout=Cout),
        grid=(nt,),
        in_specs=[x_spec, x_spec, w_spec, w_spec, allstat_spec, allstat_spec,
                  row_spec, row_spec, row_spec, row_spec],
        out_specs=[out_spec, out_spec],
        out_shape=[
            jax.ShapeDtypeStruct((M, Cout), jnp.float32),
            jax.ShapeDtypeStruct((M, Cout), jnp.float32),
        ],
        compiler_params=pltpu.CompilerParams(
            dimension_semantics=("parallel",),
            vmem_limit_bytes=_VMEM_LIMIT),
    )(xr, xi, wr, wi, ysum, ysq, gr, gi, betar, betai)

    def to_nchw(v):
        return v.reshape(N, H, W, Cout).transpose(0, 3, 1, 2)

    return {"real": to_nchw(o_real), "imag": to_nchw(o_imag)}


# speedup vs baseline: 1.0511x; 1.0215x over previous
import functools
import math

import jax
import jax.numpy as jnp
from jax.experimental import pallas as pl
from jax.experimental.pallas import tpu as pltpu

_EPS = 1e-5
_INV_SQRT2 = 1.0 / math.sqrt(2.0)
_VMEM_LIMIT = 60000 * 1024
_TM1 = 16384
_TM = 8192


def _pack_w(wr_ref, wi_ref):
    return jnp.concatenate(
        [jnp.concatenate([wr_ref[...], wi_ref[...]], axis=1),
         jnp.concatenate([-wi_ref[...], wr_ref[...]], axis=1)], axis=0)


def _stats_kernel(xr_ref, xi_ref, wr_ref, wi_ref, sum_ref, sq_ref):
    xp = jnp.concatenate([xr_ref[...], xi_ref[...]], axis=1)
    y = jnp.dot(xp, _pack_w(wr_ref, wi_ref),
                preferred_element_type=jnp.float32)
    sum_ref[...] = jnp.sum(y, axis=0, keepdims=True)[None]
    sq_ref[...] = jnp.sum(y * y, axis=0, keepdims=True)[None]


def _bn_gelu_kernel(xr_ref, xi_ref, wr_ref, wi_ref, sum_ref, sq_ref,
                    gr_ref, gi_ref, br_ref, bi_ref, or_ref, oi_ref,
                    *, inv_m, cout):
    xp = jnp.concatenate([xr_ref[...], xi_ref[...]], axis=1)
    y = jnp.dot(xp, _pack_w(wr_ref, wi_ref),
                preferred_element_type=jnp.float32)
    mean = jnp.sum(sum_ref[...], axis=0) * inv_m
    var = jnp.maximum(jnp.sum(sq_ref[...], axis=0) * inv_m - mean * mean, 0.0)
    g = jnp.concatenate([gr_ref[...], gi_ref[...]], axis=1)
    b = jnp.concatenate([br_ref[...], bi_ref[...]], axis=1)
    scale = jax.lax.rsqrt(var + _EPS) * g
    shift = b - mean * scale
    z = y * scale + shift
    o = 0.5 * z * (1.0 + jax.lax.erf(z * _INV_SQRT2))
    or_ref[...] = o[:, :cout]
    oi_ref[...] = o[:, cout:]


def kernel(x_real, x_imag, wr, wi, br, bi, gr, betar, gi, betai):
    N, Cin, H, W = x_real.shape
    Cout = wr.shape[1]
    M = N * H * W
    C2 = 2 * Cout
    nt1 = M // _TM1
    nt = M // _TM

    xr = x_real.transpose(0, 2, 3, 1).reshape(M, Cin)
    xi = x_imag.transpose(0, 2, 3, 1).reshape(M, Cin)

    x1_spec = pl.BlockSpec((_TM1, Cin), lambda i: (i, 0))
    x_spec = pl.BlockSpec((_TM, Cin), lambda i: (i, 0))
    w_spec = pl.BlockSpec((Cin, Cout), lambda i: (0, 0))
    row_spec = pl.BlockSpec((1, Cout), lambda i: (0, 0))
    stat_spec = pl.BlockSpec((1, 1, C2), lambda i: (i, 0, 0))

    ysum, ysq = pl.pallas_call(
        _stats_kernel,
        grid=(nt1,),
        in_specs=[x1_spec, x1_spec, w_spec, w_spec],
        out_specs=[stat_spec, stat_spec],
        out_shape=[
            jax.ShapeDtypeStruct((nt1, 1, C2), jnp.float32),
            jax.ShapeDtypeStruct((nt1, 1, C2), jnp.float32),
        ],
        compiler_params=pltpu.CompilerParams(
            dimension_semantics=("parallel",),
            vmem_limit_bytes=_VMEM_LIMIT),
    )(xr, xi, wr, wi)

    allstat_spec = pl.BlockSpec((nt1, 1, C2), lambda i: (0, 0, 0))
    out_spec = pl.BlockSpec((_TM, Cout), lambda i: (i, 0))

    o_real, o_imag = pl.pallas_call(
        functools.partial(_bn_gelu_kernel, inv_m=1.0 / M, cout=Cout),
        grid=(nt,),
        in_specs=[x_spec, x_spec, w_spec, w_spec, allstat_spec, allstat_spec,
                  row_spec, row_spec, row_spec, row_spec],
        out_specs=[out_spec, out_spec],
        out_shape=[
            jax.ShapeDtypeStruct((M, Cout), jnp.float32),
            jax.ShapeDtypeStruct((M, Cout), jnp.float32),
        ],
        compiler_params=pltpu.CompilerParams(
            dimension_semantics=("parallel",),
            vmem_limit_bytes=_VMEM_LIMIT),
    )(xr, xi, wr, wi, ysum, ysq, gr, gi, betar, betai)

    def to_nchw(v):
        return v.reshape(N, H, W, Cout).transpose(0, 3, 1, 2)

    return {"real": to_nchw(o_real), "imag": to_nchw(o_imag)}
